# Initial kernel scaffold; baseline (speedup 1.0000x reference)
import jax
import jax.numpy as jnp
from jax.experimental import pallas as pl


def kernel(x):
    raise NotImplementedError("write your pallas kernel here")


# speedup vs baseline: 53.5301x; 53.5301x over previous
import functools

import jax
import jax.numpy as jnp
from jax.experimental import pallas as pl
from jax.experimental.pallas import tpu as pltpu

_KEEP_RATIO = 0.2
_LEAKY_SLOPE = 0.1


def _topk_mask_body(x_ref, o_ref, *, k):
    x = x_ref[0]
    xa = jnp.where(x >= 0, x, jnp.float32(_LEAKY_SLOPE) * x)
    bits = jax.lax.bitcast_convert_type(xa, jnp.int32)
    key = jnp.where(bits >= 0, bits, bits ^ jnp.int32(0x7FFFFFFF))
    s = key.shape[1]

    def step(i, t):
        b = 31 - i
        cand = t + jax.lax.shift_left(jnp.int32(1), b.astype(jnp.int32))
        cnt = jnp.sum((key >= cand).astype(jnp.int32), axis=0, keepdims=True)
        return jnp.where(cnt >= k, cand, t)

    t0 = jnp.full((1, s), jnp.iinfo(jnp.int32).min, dtype=jnp.int32)
    t = jax.lax.fori_loop(0, 32, step, t0)
    o_ref[0] = jnp.where(key >= t, xa, jnp.float32(0.0))


def kernel(x):
    n, c, h, w = x.shape
    k = max(1, int(c * _KEEP_RATIO))
    s = h * w
    xr = x.reshape(n, c, s)
    y = pl.pallas_call(
        functools.partial(_topk_mask_body, k=k),
        grid=(n,),
        in_specs=[pl.BlockSpec((1, c, s), lambda i: (i, 0, 0))],
        out_specs=pl.BlockSpec((1, c, s), lambda i: (i, 0, 0)),
        out_shape=jax.ShapeDtypeStruct((n, c, s), jnp.float32),
        compiler_params=pltpu.CompilerParams(
            dimension_semantics=("arbitrary",),
        ),
    )(xr)
    return y.reshape(n, c, h, w)
